# baseline (speedup 1.0000x reference)
import functools
import math

import jax
import jax.numpy as jnp
from jax import lax
from jax.experimental import pallas as pl
from jax.experimental.pallas import tpu as pltpu
from jax.experimental.pallas import tpu_sc as plsc

VOC_SIZE = 1000000
SIZE = 64
MAX_LEN = 200
B = 4096
L = 200
DIVS = 10000.0
SCALE = math.sqrt(SIZE)


def _pos_enc_table():
    pos = jnp.arange(MAX_LEN, dtype=jnp.float32)[:, None]
    loc_even = jnp.arange(0, SIZE, 2, dtype=jnp.float32)[None, :]
    even_vals = jnp.sin(pos / (DIVS ** (2.0 * loc_even / SIZE)))
    odd_vals = jnp.cos(pos / (DIVS ** (2.0 * (loc_even + 1.0) / SIZE)))
    out = jnp.zeros((MAX_LEN, SIZE), dtype=jnp.float32)
    out = out.at[:, 0::2].set(even_vals)
    out = out.at[:, 1::2].set(odd_vals)
    return out


def _make_sc_kernel():
    info = plsc.get_sparse_core_info()
    nc, ns, lanes = info.num_cores, info.num_subcores, info.num_lanes
    nw = nc * ns
    total = B * L
    per_w = total // nw
    n_seq = per_w // L
    mesh = plsc.VectorSubcoreMesh(
        core_axis_name="c", subcore_axis_name="s",
        num_cores=nc, num_subcores=ns)

    @functools.partial(
        pl.kernel,
        out_type=jax.ShapeDtypeStruct((total, SIZE), jnp.float32),
        mesh=mesh,
        compiler_params=pltpu.CompilerParams(use_tc_tiling_on_sc=False),
        scratch_types=[
            pltpu.VMEM((L,), jnp.int32),
            pltpu.VMEM((L, SIZE), jnp.float32),
            pltpu.VMEM((L, SIZE), jnp.float32),
            pltpu.SemaphoreType.DMA,
        ],
    )
    def k(idx_hbm, table_hbm, pe_hbm, out_hbm, idx_v, rows_v, pe_v, sem):
        wid = lax.axis_index("s") * nc + lax.axis_index("c")
        base = wid * per_w
        pltpu.sync_copy(pe_hbm, pe_v)

        def seq_body(s, carry):
            off = base + s * L
            pltpu.sync_copy(idx_hbm.at[pl.ds(off, L)], idx_v)
            pltpu.async_copy(table_hbm.at[idx_v], rows_v, sem).wait()

            def row_body(r, c2):
                for cidx in range(SIZE // lanes):
                    sl = pl.ds(cidx * lanes, lanes)
                    rows_v[r, sl] = rows_v[r, sl] * SCALE + pe_v[r, sl]
                return c2

            lax.fori_loop(0, L, row_body, 0, unroll=2)
            pltpu.sync_copy(rows_v, out_hbm.at[pl.ds(off, L)])
            return carry

        lax.fori_loop(0, n_seq, seq_body, 0)

    return k


def kernel(enc_out, table):
    idx = enc_out.reshape(-1).astype(jnp.int32)
    pe = _pos_enc_table()
    k = _make_sc_kernel()
    out = k(idx, table, pe)
    return out.reshape(B, L, SIZE)
